# baseline (speedup 1.0000x reference)
import functools

import jax
import jax.numpy as jnp
from jax import lax
from jax.experimental import pallas as pl
from jax.experimental.pallas import tpu as pltpu
from jax.experimental.pallas import tpu_sc as plsc

_B = 16384
_D = 32
_ROWS = 1000000

_info = plsc.get_sparse_core_info()
_NC = _info.num_cores
_NS = _info.num_subcores
_NW = _NC * _NS

_CH = 1664
_NCHUNK = 601
_CPW = 19
_SPAN = _CPW * _CH
_MCAP = 1040
_CCAP = 128

_mesh = plsc.VectorSubcoreMesh(core_axis_name="c", subcore_axis_name="s")


@functools.partial(
    pl.kernel,
    mesh=_mesh,
    out_type=jax.ShapeDtypeStruct((_B * _D,), jnp.float32),
    scratch_types=[
        pltpu.VMEM((_B,), jnp.int32),
        pltpu.VMEM((_MCAP,), jnp.int32),
        pltpu.VMEM((_MCAP,), jnp.int32),
        pltpu.VMEM((_CCAP,), jnp.int32),
        pltpu.VMEM((_CCAP,), jnp.int32),
        pltpu.VMEM((32,), jnp.int32),
        pltpu.VMEM((2, _D, _CH), jnp.float32),
        pltpu.VMEM((_CCAP * _D,), jnp.float32),
        pltpu.SemaphoreType.DMA,
        pltpu.SemaphoreType.DMA,
    ],
    compiler_params=pltpu.CompilerParams(needs_layout_passes=False),
)
def _gather(idx_hbm, tableT_hbm, out_hbm, idx_v, mi_v, mg_v, ci_v, cg_v,
            bnc_v, slab_v, rows_v, sem_i, sem):
    wid = lax.axis_index("s") * _NC + lax.axis_index("c")
    lo = wid * _SPAN
    lo16 = jnp.full((16,), lo, jnp.int32)
    hi16 = lo16 + _SPAN
    icopy = pltpu.async_copy(idx_hbm, idx_v, sem_i)

    def _issue(cid, buf, ok=True):
        @pl.when(jnp.logical_and(cid < _NCHUNK, ok))
        def _go():
            pltpu.async_copy(
                tableT_hbm.at[:, :, pl.ds(pl.multiple_of(cid * _CH, 128), _CH)],
                slab_v.at[pl.ds(buf, 1)], sem,
            )

    _issue(wid * _CPW, jnp.int32(0))
    icopy.wait()

    lanes = lax.iota(jnp.int32, 16)

    def _pack_step(iv, gv, m, dst_i, dst_g, cap, carry):
        n, pend_i, pend_g = carry
        cnt = plsc.all_reduce_population_count(m)[0]
        key = jnp.where(m, lanes, 100 + lanes)
        _, perm = plsc.sort_key_val(key, lanes)
        bnc_v[pl.ds(0, 16)] = iv
        bnc_v[pl.ds(16, 16)] = gv
        comp_i = plsc.load_gather(bnc_v, [perm])
        comp_g = plsc.load_gather(bnc_v, [perm + 16])
        nf = lax.rem(n, 16)
        nf16 = jnp.full((16,), nf, jnp.int32)
        bnc_v[pl.ds(0, 16)] = pend_i
        bnc_v[pl.ds(16, 16)] = comp_i
        sel = jnp.where(lanes < nf16, lanes, 16 + lanes - nf16)
        over = jnp.minimum(16 + lanes + (16 - nf16), 31)
        merged_i = plsc.load_gather(bnc_v, [sel])
        over_i = plsc.load_gather(bnc_v, [over])
        bnc_v[pl.ds(0, 16)] = pend_g
        bnc_v[pl.ds(16, 16)] = comp_g
        merged_g = plsc.load_gather(bnc_v, [sel])
        over_g = plsc.load_gather(bnc_v, [over])
        base = jnp.minimum(n - nf, cap - 16)
        full = nf + cnt >= 16

        @pl.when(full)
        def _flush():
            dst_i[pl.ds(base, 16)] = merged_i
            dst_g[pl.ds(base, 16)] = merged_g

        full16 = jnp.full((16,), full, jnp.bool_)
        new_pend_i = jnp.where(full16, over_i, merged_i)
        new_pend_g = jnp.where(full16, over_g, merged_g)
        return (n + cnt, new_pend_i, new_pend_g)

    def _flush_tail(dst_i, dst_g, cap, carry):
        n, pend_i, pend_g = carry
        base = jnp.minimum(n - lax.rem(n, 16), cap - 16)

        @pl.when(lax.rem(n, 16) > 0)
        def _tail():
            dst_i[pl.ds(base, 16)] = pend_i
            dst_g[pl.ds(base, 16)] = pend_g

    def _scan_all(g, carry):
        iv = idx_v[pl.ds(g * 16, 16)]
        m = jnp.logical_and(iv >= lo16, iv < hi16)
        return _pack_step(iv, g * 16 + lanes, m, mi_v, mg_v, _MCAP, carry)

    zero16 = jnp.zeros((16,), jnp.int32)
    carry1 = lax.fori_loop(0, _B // 16, _scan_all,
                           (jnp.int32(0), zero16, zero16))
    _flush_tail(mi_v, mg_v, _MCAP, carry1)
    nmine = jnp.minimum(carry1[0], _MCAP - 16)
    nmine16 = jnp.full((16,), nmine, jnp.int32)

    def _chunk(c, carry):
        cid = wid * _CPW + c
        clo = cid * _CH
        par = lax.rem(c, 2)

        @pl.when(cid < _NCHUNK)
        def _wait_mine():
            pltpu.make_async_copy(
                tableT_hbm.at[:, :, pl.ds(0, _CH)],
                slab_v.at[pl.ds(par, 1)], sem,
            ).wait()

        _issue(cid + 1, 1 - par, c + 1 < _CPW)

        clo16 = jnp.full((16,), clo, jnp.int32)
        chi16 = clo16 + _CH

        def _scan_mine(k, carry2):
            iv = mi_v[pl.ds(k * 16, 16)]
            gv = mg_v[pl.ds(k * 16, 16)]
            valid = (k * 16 + lanes) < nmine16
            m = jnp.logical_and(valid,
                                jnp.logical_and(iv >= clo16, iv < chi16))
            return _pack_step(iv, gv, m, ci_v, cg_v, _CCAP, carry2)

        carry2 = lax.fori_loop(0, (nmine + 15) // 16, _scan_mine,
                               (jnp.int32(0), zero16, zero16))
        _flush_tail(ci_v, cg_v, _CCAP, carry2)
        nch = jnp.minimum(carry2[0], _CCAP - 16)

        def _emit(k, _2):
            iv = ci_v[pl.ds(k * 16, 16)]
            gv = cg_v[pl.ds(k * 16, 16)]
            col16 = iv - clo16
            for l in range(16):
                @pl.when(k * 16 + l < nch)
                def _one(l=l):
                    s2 = k * 16 + l
                    col = jnp.full((16,), col16[l], jnp.int32)
                    par16 = jnp.full((16,), par, jnp.int32)
                    v0 = plsc.load_gather(slab_v, [par16, lanes, col])
                    v1 = plsc.load_gather(slab_v, [par16, lanes + 16, col])
                    rows_v[pl.ds(s2 * _D, 16)] = v0
                    rows_v[pl.ds(s2 * _D + 16, 16)] = v1
                    g = gv[l]
                    pltpu.async_copy(
                        rows_v.at[pl.ds(s2 * _D, _D)],
                        out_hbm.at[pl.ds(g * _D, _D)],
                        sem_i,
                    )
            return _2

        lax.fori_loop(0, (nch + 15) // 16, _emit, jnp.int32(0))

        def _drain(k, _3):
            pltpu.make_async_copy(
                out_hbm.at[pl.ds(0, _D)], rows_v.at[pl.ds(0, _D)], sem_i
            ).wait()
            return _3

        lax.fori_loop(0, nch, _drain, jnp.int32(0))
        return carry

    lax.fori_loop(0, _CPW, _chunk, jnp.int32(0))


def kernel(x, ker):
    out1 = _gather(x.astype(jnp.int32), ker.T.reshape(1, _D, _ROWS))
    return out1.reshape(_B, _D)[:, :, None]
